# baseline (speedup 1.0000x reference)
import functools

import jax
import jax.numpy as jnp
from jax import lax
from jax.experimental import pallas as pl
from jax.experimental.pallas import tpu as pltpu
from jax.experimental.pallas import tpu_sc as plsc

_VOCAB = 100000
_D = 2048
_B = 8192
_NC = 2
_NS = 16
_NW = _NC * _NS
_BPW = _B // _NW
_C = 8
_NCHUNK = _BPW // _C
_NBUF = 7
_LA = 4


def _body(idx_hbm, tab_hbm, out_hbm, idx_v, *rest):
    bufs = rest[:_NBUF]
    gsem = rest[_NBUF:2 * _NBUF]
    ssem = rest[2 * _NBUF:3 * _NBUF]
    wid = lax.axis_index("s") * _NC + lax.axis_index("c")
    base = wid * _BPW

    pltpu.sync_copy(idx_hbm.at[wid], idx_v)

    gh = [None] * _NBUF
    sh = [None] * _NBUF

    for c in range(_LA):
        gh[c % _NBUF] = pltpu.async_copy(
            tab_hbm.at[idx_v.at[c]], bufs[c % _NBUF], gsem[c % _NBUF])

    for j in range(_NCHUNK):
        s = j % _NBUF
        gh[s].wait()
        sh[s] = pltpu.async_copy(
            bufs[s], out_hbm.at[pl.ds(base + j * _C, _C)], ssem[s])
        c = j + _LA
        if c < _NCHUNK:
            cs = c % _NBUF
            if sh[cs] is not None:
                sh[cs].wait()
            gh[cs] = pltpu.async_copy(
                tab_hbm.at[idx_v.at[c]], bufs[cs], gsem[cs])

    for s in range(_NBUF):
        if sh[s] is not None:
            sh[s].wait()


@jax.jit
def _gather(idx, wte):
    run = pl.kernel(
        _body,
        out_type=jax.ShapeDtypeStruct((_B, _D), jnp.float32),
        mesh=plsc.VectorSubcoreMesh(core_axis_name="c", subcore_axis_name="s"),
        scratch_types=(
            [pltpu.VMEM((_NCHUNK, _C), jnp.int32)]
            + [pltpu.VMEM((_C, _D), jnp.float32) for _ in range(_NBUF)]
            + [pltpu.SemaphoreType.DMA for _ in range(2 * _NBUF)]
        ),
    )
    return run(idx, wte)


def kernel(ipt, wte):
    idx = ipt.astype(jnp.int32).reshape(_NW, _NCHUNK, _C)
    out = _gather(idx, wte)
    return out.reshape(ipt.shape[0], ipt.shape[1], _D)


# speedup vs baseline: 1.0081x; 1.0081x over previous
import functools

import jax
import jax.numpy as jnp
from jax import lax
from jax.experimental import pallas as pl
from jax.experimental.pallas import tpu as pltpu
from jax.experimental.pallas import tpu_sc as plsc

_VOCAB = 100000
_D = 2048
_B = 8192
_NC = 2
_NS = 16
_NW = _NC * _NS
_BPW = _B // _NW
_C = 8
_NCHUNK = _BPW // _C
_NBUF = 7
_LA = 6


def _body(idx_hbm, tab_hbm, out_hbm, idx_v, *rest):
    bufs = rest[:_NBUF]
    gsem = rest[_NBUF:2 * _NBUF]
    ssem = rest[2 * _NBUF:3 * _NBUF]
    wid = lax.axis_index("s") * _NC + lax.axis_index("c")
    base = wid * _BPW

    pltpu.sync_copy(idx_hbm.at[wid], idx_v)

    gh = [None] * _NBUF
    sh = [None] * _NBUF

    for c in range(_LA):
        gh[c % _NBUF] = pltpu.async_copy(
            tab_hbm.at[idx_v.at[c]], bufs[c % _NBUF], gsem[c % _NBUF])

    for j in range(_NCHUNK):
        s = j % _NBUF
        gh[s].wait()
        sh[s] = pltpu.async_copy(
            bufs[s], out_hbm.at[pl.ds(base + j * _C, _C)], ssem[s])
        c = j + _LA
        if c < _NCHUNK:
            cs = c % _NBUF
            if sh[cs] is not None:
                sh[cs].wait()
            gh[cs] = pltpu.async_copy(
                tab_hbm.at[idx_v.at[c]], bufs[cs], gsem[cs])

    for s in range(_NBUF):
        if sh[s] is not None:
            sh[s].wait()


@jax.jit
def _gather(idx, wte):
    run = pl.kernel(
        _body,
        out_type=jax.ShapeDtypeStruct((_B, _D), jnp.float32),
        mesh=plsc.VectorSubcoreMesh(core_axis_name="c", subcore_axis_name="s"),
        scratch_types=(
            [pltpu.VMEM((_NCHUNK, _C), jnp.int32)]
            + [pltpu.VMEM((_C, _D), jnp.float32) for _ in range(_NBUF)]
            + [pltpu.SemaphoreType.DMA for _ in range(2 * _NBUF)]
        ),
    )
    return run(idx, wte)


def kernel(ipt, wte):
    idx = ipt.astype(jnp.int32).reshape(_NW, _NCHUNK, _C)
    out = _gather(idx, wte)
    return out.reshape(ipt.shape[0], ipt.shape[1], _D)


# speedup vs baseline: 1.0116x; 1.0035x over previous
import functools

import jax
import jax.numpy as jnp
from jax import lax
from jax.experimental import pallas as pl
from jax.experimental.pallas import tpu as pltpu
from jax.experimental.pallas import tpu_sc as plsc

_VOCAB = 100000
_D = 2048
_B = 8192
_NC = 2
_NS = 16
_NW = _NC * _NS
_BPW = _B // _NW
_C = 8
_NCHUNK = _BPW // _C
_NBUF = 7
_LA = 5


def _body(idx_hbm, tab_hbm, out_hbm, idx_v, *rest):
    bufs = rest[:_NBUF]
    gsem = rest[_NBUF:2 * _NBUF]
    ssem = rest[2 * _NBUF:3 * _NBUF]
    wid = lax.axis_index("s") * _NC + lax.axis_index("c")
    base = wid * _BPW

    pltpu.sync_copy(idx_hbm.at[wid], idx_v)

    gh = [None] * _NBUF
    sh = [None] * _NBUF

    for c in range(_LA):
        gh[c % _NBUF] = pltpu.async_copy(
            tab_hbm.at[idx_v.at[c]], bufs[c % _NBUF], gsem[c % _NBUF])

    for j in range(_NCHUNK):
        s = j % _NBUF
        gh[s].wait()
        sh[s] = pltpu.async_copy(
            bufs[s], out_hbm.at[pl.ds(base + j * _C, _C)], ssem[s])
        c = j + _LA
        if c < _NCHUNK:
            cs = c % _NBUF
            if sh[cs] is not None:
                sh[cs].wait()
            gh[cs] = pltpu.async_copy(
                tab_hbm.at[idx_v.at[c]], bufs[cs], gsem[cs])

    for s in range(_NBUF):
        if sh[s] is not None:
            sh[s].wait()


@jax.jit
def _gather(idx, wte):
    run = pl.kernel(
        _body,
        out_type=jax.ShapeDtypeStruct((_B, _D), jnp.float32),
        mesh=plsc.VectorSubcoreMesh(core_axis_name="c", subcore_axis_name="s"),
        scratch_types=(
            [pltpu.VMEM((_NCHUNK, _C), jnp.int32)]
            + [pltpu.VMEM((_C, _D), jnp.float32) for _ in range(_NBUF)]
            + [pltpu.SemaphoreType.DMA for _ in range(2 * _NBUF)]
        ),
    )
    return run(idx, wte)


def kernel(ipt, wte):
    idx = ipt.astype(jnp.int32).reshape(_NW, _NCHUNK, _C)
    out = _gather(idx, wte)
    return out.reshape(ipt.shape[0], ipt.shape[1], _D)
